# baseline (speedup 1.0000x reference)
import jax
import jax.numpy as jnp
from jax.experimental import pallas as pl
from jax.experimental.pallas import tpu as pltpu

B = 16
HW = 1024
K = 1024
D = 256


def _vq_kernel(z_ref, w2_ref, zsq_ref, wsq_ref, out_ref):
    z = z_ref[0]
    w2 = w2_ref[...]
    zsq = zsq_ref[0, 0]
    wsq = wsq_ref[...]
    mm2 = jax.lax.dot_general(
        z, w2, (((1,), (1,)), ((), ())),
        preferred_element_type=jnp.float32)
    d = (zsq[:, None] + wsq[None, :]) - mm2
    idx = jnp.argmin(d, axis=1).astype(jnp.int32)
    out_ref[0, 0, :] = idx


def kernel(z_e, W):
    zsq = jnp.sum(z_e ** 2, axis=-1).reshape(B, 1, HW)
    wsq = jnp.sum(W ** 2, axis=1)
    w2 = W + W
    out = pl.pallas_call(
        _vq_kernel,
        grid=(B,),
        in_specs=[
            pl.BlockSpec((1, HW, D), lambda b: (b, 0, 0)),
            pl.BlockSpec((K, D), lambda b: (0, 0)),
            pl.BlockSpec((1, 1, HW), lambda b: (b, 0, 0)),
            pl.BlockSpec((K,), lambda b: (0,)),
        ],
        out_specs=pl.BlockSpec((1, 1, HW), lambda b: (b, 0, 0)),
        out_shape=jax.ShapeDtypeStruct((B, 1, HW), jnp.int32),
        compiler_params=pltpu.CompilerParams(
            dimension_semantics=("parallel",)),
    )(z_e, w2, zsq, wsq)
    return out.reshape(B, HW)
